# baseline (speedup 1.0000x reference)
import functools

import jax
import jax.numpy as jnp
from jax.experimental import pallas as pl
from jax.experimental.pallas import tpu as pltpu

_BM = 200
_XC = 4


def _gcn_body(w_ref, b_ref, x_ref, adj_ref, out_ref, xbuf_ref, xw_ref,
              xsem_ref):
    n = adj_ref.shape[1]
    xrows = n // _XC

    def x_copy(c):
        return pltpu.make_async_copy(
            x_ref.at[pl.ds(c * xrows, xrows), :],
            xbuf_ref.at[pl.ds(c * xrows, xrows), :],
            xsem_ref.at[c],
        )

    @pl.when(pl.program_id(0) == 0)
    def _():
        for c in range(_XC):
            x_copy(c).start()
        for c in range(_XC):
            x_copy(c).wait()
        xw_ref[...] = jnp.dot(
            xbuf_ref[...], w_ref[...], preferred_element_type=jnp.float32
        )

    i = pl.program_id(0)
    out_ref[pl.ds(i * _BM, _BM), :] = (
        jnp.dot(adj_ref[...], xw_ref[...], preferred_element_type=jnp.float32)
        + b_ref[...]
    )


@functools.partial(jax.jit, static_argnames=())
def kernel(x, adj, w, b):
    n, f = x.shape
    h = w.shape[1]

    out = pl.pallas_call(
        _gcn_body,
        grid=(n // _BM,),
        in_specs=[
            pl.BlockSpec((f, h), lambda i: (0, 0)),
            pl.BlockSpec((1, h), lambda i: (0, 0)),
            pl.BlockSpec(memory_space=pl.ANY),
            pl.BlockSpec((_BM, n), lambda i: (i, 0)),
        ],
        out_specs=pl.BlockSpec((n, h), lambda i: (0, 0)),
        out_shape=jax.ShapeDtypeStruct((n, h), jnp.float32),
        scratch_shapes=[
            pltpu.VMEM((n, f), jnp.float32),
            pltpu.VMEM((n, h), jnp.float32),
            pltpu.SemaphoreType.DMA((_XC,)),
        ],
    )(w, b.reshape(1, h), x, adj)
    return out


# speedup vs baseline: 1.0163x; 1.0163x over previous
import functools

import jax
import jax.numpy as jnp
from jax.experimental import pallas as pl
from jax.experimental.pallas import tpu as pltpu

_BM = 200
_GROUP = 5


def _gcn_body(x_ref, w_ref, b_ref, adj_ref, out_ref, xw_ref):
    @pl.when(pl.program_id(0) == 0)
    def _():
        xw_ref[...] = jnp.dot(
            x_ref[...], w_ref[...], preferred_element_type=jnp.float32
        )

    j = jax.lax.rem(pl.program_id(0), _GROUP)
    out_ref[pl.ds(j * _BM, _BM), :] = (
        jnp.dot(adj_ref[...], xw_ref[...], preferred_element_type=jnp.float32)
        + b_ref[...]
    )


@functools.partial(jax.jit, static_argnames=())
def kernel(x, adj, w, b):
    n, f = x.shape
    h = w.shape[1]

    out = pl.pallas_call(
        _gcn_body,
        grid=(n // _BM,),
        in_specs=[
            pl.BlockSpec((n, f), lambda i: (0, 0)),
            pl.BlockSpec((f, h), lambda i: (0, 0)),
            pl.BlockSpec((1, h), lambda i: (0, 0)),
            pl.BlockSpec((_BM, n), lambda i: (i, 0)),
        ],
        out_specs=pl.BlockSpec((_GROUP * _BM, h), lambda i: (i // _GROUP, 0)),
        out_shape=jax.ShapeDtypeStruct((n, h), jnp.float32),
        scratch_shapes=[pltpu.VMEM((n, h), jnp.float32)],
    )(x, w, b.reshape(1, h), adj)
    return out


# speedup vs baseline: 1.0194x; 1.0031x over previous
import functools

import jax
import jax.numpy as jnp
from jax.experimental import pallas as pl
from jax.experimental.pallas import tpu as pltpu

_BM = 200
_HPAD = 128


def _gcn_body(x_ref, w_ref, b_ref, adj_ref, out_ref, xw_ref):
    @pl.when(pl.program_id(0) == 0)
    def _():
        xw_ref[...] = jnp.dot(
            x_ref[...], w_ref[...], preferred_element_type=jnp.float32
        )

    h = w_ref.shape[1]
    out_ref[:, :h] = (
        jnp.dot(adj_ref[...], xw_ref[...], preferred_element_type=jnp.float32)
        + b_ref[...]
    )


@functools.partial(jax.jit, static_argnames=())
def kernel(x, adj, w, b):
    n, f = x.shape
    h = w.shape[1]

    out = pl.pallas_call(
        _gcn_body,
        grid=(n // _BM,),
        in_specs=[
            pl.BlockSpec((n, f), lambda i: (0, 0)),
            pl.BlockSpec((f, h), lambda i: (0, 0)),
            pl.BlockSpec((1, h), lambda i: (0, 0)),
            pl.BlockSpec((_BM, n), lambda i: (i, 0)),
        ],
        out_specs=pl.BlockSpec((_BM, _HPAD), lambda i: (i, 0)),
        out_shape=jax.ShapeDtypeStruct((n, _HPAD), jnp.float32),
        scratch_shapes=[pltpu.VMEM((n, h), jnp.float32)],
    )(x, w, b.reshape(1, h), adj)
    return out[:, :h]
